# Initial kernel scaffold; baseline (speedup 1.0000x reference)
import jax
import jax.numpy as jnp
from jax.experimental import pallas as pl


def kernel(x, edge_index, edge_attr, batch, We, be, W1, b1, g1, bt1, W2, b2, eps, Wp, bp):
    raise NotImplementedError("write your pallas kernel here")


# speedup vs baseline: 2.1856x; 2.1856x over previous
import functools

import jax
import jax.numpy as jnp
from jax import lax
from jax.experimental import pallas as pl
from jax.experimental.pallas import tpu as pltpu
from jax.experimental.pallas import tpu_sc as plsc

NUM_LAYERS = 5
EMB = 81
D = 96
N = 50000
NG = 512
E = 800000
C = 128
U = 64
R = 784
N2 = U * R
E_PAD = 802816
NW = 32
EPT = E_PAD // NW
KD = 16
RS = 1.0 / (1.0 + 1e-5) ** 0.5

_MESH = plsc.VectorSubcoreMesh(
    core_axis_name="c", subcore_axis_name="s", num_cores=2, num_subcores=16)
_SC_PARAMS = pltpu.CompilerParams(use_tc_tiling_on_sc=False,
                                 needs_layout_passes=False)


def _wid():
  return lax.axis_index("s") * 2 + lax.axis_index("c")


@functools.partial(
    pl.kernel,
    out_type=(jax.ShapeDtypeStruct((E_PAD,), jnp.int32),
              jax.ShapeDtypeStruct((E_PAD, KD), jnp.float32)),
    mesh=_MESH,
    compiler_params=_SC_PARAMS,
    scratch_types=[
        pltpu.VMEM((C,), jnp.int32),
        pltpu.VMEM((C,), jnp.int32),
        pltpu.VMEM((C, KD), jnp.float32),
        pltpu.SemaphoreType.DMA,
    ],
)
def _prep(src_hbm, ea_hbm, order_hbm, srcs_out, eas_out, oidx_v, sv, eav, sem):
  base = _wid() * EPT

  def chunk(c, carry):
    cs = pl.multiple_of(base + c * C, 8)
    pltpu.sync_copy(order_hbm.at[pl.ds(cs, C)], oidx_v)
    pltpu.async_copy(src_hbm.at[oidx_v], sv, sem).wait()
    pltpu.async_copy(ea_hbm.at[oidx_v], eav, sem).wait()
    pltpu.sync_copy(sv, srcs_out.at[pl.ds(cs, C)])
    pltpu.sync_copy(eav, eas_out.at[pl.ds(cs, C)])
    return carry

  lax.fori_loop(0, EPT // C, chunk, 0)


RP = R + 1
ZR = 157


@functools.partial(
    pl.kernel,
    out_type=jax.ShapeDtypeStruct((N2, D), jnp.float32),
    mesh=_MESH,
    compiler_params=_SC_PARAMS,
    scratch_types=[
        pltpu.VMEM_SHARED((16 * RP, D), jnp.float32),
        pltpu.VMEM((C, D), jnp.float32),
        pltpu.VMEM((C, D), jnp.float32),
        pltpu.VMEM((C,), jnp.int32),
        pltpu.VMEM((C,), jnp.int32),
        pltpu.VMEM((C,), jnp.int32),
        pltpu.VMEM((ZR, D), jnp.float32),
        pltpu.VMEM((U + 16,), jnp.int32),
        pltpu.VMEM((U + 16,), jnp.int32),
        pltpu.SemaphoreType.DMA,
    ],
)
def _edge(hx, srcs, dsts, embs, es_hbm, ee_hbm, aggr,
          shacc, rows_v, emb_v, idx_v, dst_v, scat_v, zb_v, es_v, ee_v, sem):
  wid = _wid()
  sid = lax.axis_index("s")
  tile_base = sid * RP
  pltpu.sync_copy(es_hbm, es_v)
  pltpu.sync_copy(ee_hbm, ee_v)
  zero16 = jnp.zeros((16,), jnp.float32)

  def zrow(r, carry):
    for j in range(D // 16):
      zb_v[r, pl.ds(16 * j, 16)] = zero16
    return carry

  lax.fori_loop(0, ZR, zrow, 0)

  for k in range(U // NW):
    u = wid + NW * k
    node_start = u * R

    for z in range(RP // ZR):
      pltpu.sync_copy(zb_v, shacc.at[pl.ds(tile_base + z * ZR, ZR)])

    es = es_v[pl.ds(u, 16)][0]
    ee = ee_v[pl.ds(u, 16)][0]
    e_lo = es & ~7
    n_ch = (ee - e_lo + (C - 1)) >> 7

    def chunk(c, carry):
      cs = pl.multiple_of(e_lo + c * C, 8)
      pltpu.sync_copy(srcs.at[pl.ds(cs, C)], idx_v)
      pltpu.sync_copy(dsts.at[pl.ds(cs, C)], dst_v)
      pltpu.sync_copy(embs.at[pl.ds(cs, C)], emb_v)
      pltpu.async_copy(hx.at[idx_v], rows_v, sem).wait()

      def gidx(g, carry2):
        dvec = dst_v[pl.ds(g * 16, 16)] - node_start
        okv = jnp.logical_and(dvec >= 0, dvec < R)
        scat_v[pl.ds(g * 16, 16)] = jnp.where(okv, dvec, R) + tile_base
        return carry2

      lax.fori_loop(0, C // 16, gidx, 0)

      def emsg(e, carry2):
        for j in range(D // 16):
          sl = pl.ds(16 * j, 16)
          rows_v[e, sl] = jnp.maximum(rows_v[e, sl] + emb_v[e, sl], 0.0)
        return carry2

      lax.fori_loop(0, C, emsg, 0)
      pltpu.sync_copy(rows_v, shacc.at[scat_v], add=True)
      return carry

    lax.fori_loop(0, n_ch, chunk, 0)
    pltpu.sync_copy(shacc.at[pl.ds(tile_base, R)],
                    aggr.at[pl.ds(pl.multiple_of(node_start, 8), R)])


@functools.partial(
    pl.kernel,
    out_type=jax.ShapeDtypeStruct((NG,), jnp.float32),
    mesh=_MESH,
    compiler_params=_SC_PARAMS,
    scratch_types=[
        pltpu.VMEM_SHARED((16 * 17, D), jnp.float32),
        pltpu.VMEM((17, D), jnp.float32),
        pltpu.VMEM((C, D), jnp.float32),
        pltpu.VMEM((D,), jnp.float32),
        pltpu.VMEM((16,), jnp.float32),
        pltpu.VMEM((C,), jnp.int32),
        pltpu.VMEM((C,), jnp.int32),
        pltpu.VMEM((NW + 16,), jnp.int32),
        pltpu.VMEM((NW + 16,), jnp.int32),
        pltpu.VMEM((16,), jnp.float32),
        pltpu.SemaphoreType.DMA,
    ],
)
def _pool(hfin, batchp, ns_hbm, ne_hbm, wp_hbm, bp_hbm, out,
          shacc, accum, rows_v, wp_v, out_v, b_v, scat_v, ns_v, ne_v, bp_v,
          sem):
  wid = _wid()
  sid = lax.axis_index("s")
  tile_base = sid * 17
  pltpu.sync_copy(ns_hbm, ns_v)
  pltpu.sync_copy(ne_hbm, ne_v)
  pltpu.sync_copy(bp_hbm, bp_v)
  pltpu.sync_copy(wp_hbm, wp_v)
  g0 = wid * 16
  zero16 = jnp.zeros((16,), jnp.float32)

  def zrow(r, carry):
    for j in range(D // 16):
      accum[r, pl.ds(16 * j, 16)] = zero16
    return carry

  lax.fori_loop(0, 17, zrow, 0)
  pltpu.sync_copy(accum, shacc.at[pl.ds(tile_base, 17)])

  n_lo = ns_v[pl.ds(wid, 16)][0] & ~7
  n_ch = (ne_v[pl.ds(wid, 16)][0] - n_lo + (C - 1)) >> 7

  def chunk(c, carry):
    cs = pl.multiple_of(n_lo + c * C, 8)
    pltpu.sync_copy(hfin.at[pl.ds(cs, C)], rows_v)
    pltpu.sync_copy(batchp.at[pl.ds(cs, C)], b_v)

    def gidx(g, carry2):
      gvec = b_v[pl.ds(g * 16, 16)] - g0
      okv = jnp.logical_and(gvec >= 0, gvec < 16)
      scat_v[pl.ds(g * 16, 16)] = jnp.where(okv, gvec, 16) + tile_base
      return carry2

    lax.fori_loop(0, C // 16, gidx, 0)
    pltpu.sync_copy(rows_v, shacc.at[scat_v], add=True)
    return carry

  lax.fori_loop(0, n_ch, chunk, 0)
  pltpu.sync_copy(shacc.at[pl.ds(tile_base, 17)], accum)

  lane = lax.iota(jnp.int32, 16)
  z = bp_v[...]
  for g in range(16):
    acc = jnp.zeros((16,), jnp.float32)
    for j in range(D // 16):
      sl = pl.ds(16 * j, 16)
      acc = acc + accum[g, sl] * wp_v[sl]
    tot = jnp.sum(acc)
    z = jnp.where(lane == g, z + tot, z)
  z = jnp.minimum(jnp.maximum(z, 0.0), 50.0)
  out_v[...] = z
  pltpu.sync_copy(out_v, out.at[pl.ds(pl.multiple_of(g0, 8), 16)])


_EB = 8192


def _emb_body(ea_ref, w_ref, b_ref, out_ref):
  out_ref[...] = (
      jnp.dot(ea_ref[...], w_ref[...], preferred_element_type=jnp.float32)
      + b_ref[0:1, :])


def _emb_call(ea_s, wep, bep):
  return pl.pallas_call(
      _emb_body,
      grid=(E_PAD // _EB,),
      in_specs=[
          pl.BlockSpec((_EB, KD), lambda i: (i, 0)),
          pl.BlockSpec((KD, D), lambda i: (0, 0)),
          pl.BlockSpec((8, D), lambda i: (0, 0)),
      ],
      out_specs=pl.BlockSpec((_EB, D), lambda i: (i, 0)),
      out_shape=jax.ShapeDtypeStruct((E_PAD, D), jnp.float32),
  )(ea_s, wep, bep)


def _p1_body(h_ref, a_ref, w1_ref, p_ref, t1_ref, s_ref):
  u = h_ref[...] * p_ref[4:5, :] + a_ref[...]
  t1 = jnp.dot(u, w1_ref[...], preferred_element_type=jnp.float32) + p_ref[0:1, :]
  t1_ref[...] = t1
  s1 = jnp.sum(t1, axis=0)
  s2 = jnp.sum(t1 * t1, axis=0)
  z = jnp.zeros((6, D), jnp.float32)
  s_ref[...] = jnp.concatenate([s1[None], s2[None], z], axis=0)[None]


def _p1_call(h, aggr, w1p, p):
  return pl.pallas_call(
      _p1_body,
      grid=(U,),
      in_specs=[
          pl.BlockSpec((R, D), lambda i: (i, 0)),
          pl.BlockSpec((R, D), lambda i: (i, 0)),
          pl.BlockSpec((D, D), lambda i: (0, 0)),
          pl.BlockSpec((8, D), lambda i: (0, 0)),
      ],
      out_specs=[
          pl.BlockSpec((R, D), lambda i: (i, 0)),
          pl.BlockSpec((1, 8, D), lambda i: (i, 0, 0)),
      ],
      out_shape=[
          jax.ShapeDtypeStruct((N2, D), jnp.float32),
          jax.ShapeDtypeStruct((U, 8, D), jnp.float32),
      ],
  )(h, aggr, w1p, p)


def _p2_body(last, t1_ref, h_ref, s_ref, w2_ref, p_ref, out_ref):
  i = pl.program_id(0)
  b1 = p_ref[0:1, :]
  s = s_ref[...]
  s1 = jnp.sum(s[:, 0, :], axis=0, keepdims=True) - float(N2 - N) * b1
  s2 = jnp.sum(s[:, 1, :], axis=0, keepdims=True) - float(N2 - N) * b1 * b1
  mu = s1 / float(N)
  var = s2 / float(N) - mu * mu
  inv = lax.rsqrt(var + 1e-5)
  t = (t1_ref[...] - mu) * inv * p_ref[1:2, :] + p_ref[2:3, :]
  t = jnp.maximum(t, 0.0)
  t = jnp.dot(t, w2_ref[...], preferred_element_type=jnp.float32) + p_ref[3:4, :]
  t = jnp.maximum(t, 0.0) * RS
  if not last:
    t = jnp.maximum(t, 0.0)
  hn = t + h_ref[...]
  rid = i * R + lax.broadcasted_iota(jnp.int32, (R, D), 0)
  out_ref[...] = jnp.where(rid < N, hn, 0.0)


def _p2_call(t1, h, s, w2p, p, last):
  return pl.pallas_call(
      functools.partial(_p2_body, last),
      grid=(U,),
      in_specs=[
          pl.BlockSpec((R, D), lambda i: (i, 0)),
          pl.BlockSpec((R, D), lambda i: (i, 0)),
          pl.BlockSpec((U, 8, D), lambda i: (0, 0, 0)),
          pl.BlockSpec((D, D), lambda i: (0, 0)),
          pl.BlockSpec((8, D), lambda i: (0, 0)),
      ],
      out_specs=pl.BlockSpec((R, D), lambda i: (i, 0)),
      out_shape=jax.ShapeDtypeStruct((N2, D), jnp.float32),
  )(t1, h, s, w2p, p)


def kernel(x, edge_index, edge_attr, batch, We, be, W1, b1, g1, bt1,
           W2, b2, eps, Wp, bp):
  f32 = jnp.float32
  src = edge_index[0].astype(jnp.int32)
  dst = edge_index[1].astype(jnp.int32)

  dst_s, order = lax.sort_key_val(dst, jnp.arange(E, dtype=jnp.int32))
  dst_sp = jnp.concatenate([dst_s, jnp.full((E_PAD - E,), N2, jnp.int32)])
  order_p = jnp.concatenate([order, jnp.zeros((E_PAD - E,), jnp.int32)])
  bounds = jnp.arange(U + 1, dtype=jnp.int32) * R
  ss = jnp.searchsorted(dst_s, bounds, side="left").astype(jnp.int32)
  es = jnp.pad(ss[:U], (0, 16))
  ee = jnp.pad(ss[1:], (0, 16))

  ea16 = jnp.pad(edge_attr.astype(f32), ((0, 0), (0, KD - 6)))
  src_s, ea_s = _prep(src, ea16, order_p)

  Wep = jnp.zeros((NUM_LAYERS, KD, D), f32).at[:, :6, :EMB].set(We)
  bep = jnp.zeros((NUM_LAYERS, 8, D), f32).at[:, 0, :EMB].set(be)
  W1p = jnp.zeros((NUM_LAYERS, D, D), f32).at[:, :EMB, :EMB].set(W1)
  W2p = jnp.zeros((NUM_LAYERS, D, D), f32).at[:, :EMB, :EMB].set(W2)
  P = jnp.zeros((NUM_LAYERS, 8, D), f32)
  P = P.at[:, 0, :EMB].set(b1)
  P = P.at[:, 1, :EMB].set(g1)
  P = P.at[:, 2, :EMB].set(bt1)
  P = P.at[:, 3, :EMB].set(b2)
  P = P.at[:, 4, :].set((1.0 + eps)[:, None])

  h = jnp.pad(x.astype(f32), ((0, N2 - N), (0, D - EMB)))

  for l in range(NUM_LAYERS):
    emb = _emb_call(ea_s, Wep[l], bep[l])
    aggr = _edge(h, src_s, dst_sp, emb, es, ee)
    t1, s = _p1_call(h, aggr, W1p[l], P[l])
    h = _p2_call(t1, h, s, W2p[l], P[l], l == NUM_LAYERS - 1)

  batchp = jnp.concatenate([batch.astype(jnp.int32),
                            jnp.full((N2 - N,), NG, jnp.int32)])
  gb = jnp.arange(NW + 1, dtype=jnp.int32) * 16
  nss = jnp.searchsorted(batch.astype(jnp.int32), gb, side="left").astype(jnp.int32)
  wpp = jnp.zeros((D,), f32).at[:EMB].set(Wp[:, 0])
  bp16 = jnp.broadcast_to(bp.astype(f32), (16,))
  out = _pool(h, batchp, jnp.pad(nss[:NW], (0, 16)), jnp.pad(nss[1:], (0, 16)),
              wpp, bp16)
  return out.reshape(NG, 1)


# speedup vs baseline: 2.6403x; 1.2080x over previous
import functools

import jax
import jax.numpy as jnp
from jax import lax
from jax.experimental import pallas as pl
from jax.experimental.pallas import tpu as pltpu
from jax.experimental.pallas import tpu_sc as plsc

NUM_LAYERS = 5
EMB = 81
D = 96
N = 50000
NG = 512
E = 800000
C = 128
U = 64
R = 784
N2 = U * R
E_PAD = 802816
NW = 32
EPT = E_PAD // NW
KD = 16
RS = 1.0 / (1.0 + 1e-5) ** 0.5

_MESH = plsc.VectorSubcoreMesh(
    core_axis_name="c", subcore_axis_name="s", num_cores=2, num_subcores=16)
_SC_PARAMS = pltpu.CompilerParams(use_tc_tiling_on_sc=False,
                                 needs_layout_passes=False)


def _wid():
  return lax.axis_index("s") * 2 + lax.axis_index("c")


@functools.partial(
    pl.kernel,
    out_type=(jax.ShapeDtypeStruct((E_PAD,), jnp.int32),
              jax.ShapeDtypeStruct((E_PAD, KD), jnp.float32)),
    mesh=_MESH,
    compiler_params=_SC_PARAMS,
    scratch_types=[
        pltpu.VMEM((C,), jnp.int32),
        pltpu.VMEM((C,), jnp.int32),
        pltpu.VMEM((C, KD), jnp.float32),
        pltpu.SemaphoreType.DMA,
    ],
)
def _prep(src_hbm, ea_hbm, order_hbm, srcs_out, eas_out, oidx_v, sv, eav, sem):
  base = _wid() * EPT

  def chunk(c, carry):
    cs = pl.multiple_of(base + c * C, 8)
    pltpu.sync_copy(order_hbm.at[pl.ds(cs, C)], oidx_v)
    pltpu.async_copy(src_hbm.at[oidx_v], sv, sem).wait()
    pltpu.async_copy(ea_hbm.at[oidx_v], eav, sem).wait()
    pltpu.sync_copy(sv, srcs_out.at[pl.ds(cs, C)])
    pltpu.sync_copy(eav, eas_out.at[pl.ds(cs, C)])
    return carry

  lax.fori_loop(0, EPT // C, chunk, 0)


RP = R + 1
ZR = 157
CE = 128


@functools.partial(
    pl.kernel,
    out_type=jax.ShapeDtypeStruct((N2, D), jnp.float32),
    mesh=_MESH,
    compiler_params=_SC_PARAMS,
    scratch_types=[
        pltpu.VMEM_SHARED((16 * RP, D), jnp.float32),
        pltpu.VMEM((CE, D), jnp.float32),
        pltpu.VMEM((CE, D), jnp.float32),
        pltpu.VMEM((CE, D), jnp.float32),
        pltpu.VMEM((CE, D), jnp.float32),
        pltpu.VMEM((CE,), jnp.int32),
        pltpu.VMEM((CE,), jnp.int32),
        pltpu.VMEM((CE,), jnp.int32),
        pltpu.VMEM((CE,), jnp.int32),
        pltpu.VMEM((CE,), jnp.int32),
        pltpu.VMEM((2, 128), jnp.int32),
        pltpu.VMEM((2, 128), jnp.int32),
        pltpu.VMEM((U + 16,), jnp.int32),
        pltpu.VMEM((U + 16,), jnp.int32),
        pltpu.SemaphoreType.DMA,
        pltpu.SemaphoreType.DMA,
        pltpu.SemaphoreType.DMA,
        pltpu.SemaphoreType.DMA,
        pltpu.SemaphoreType.DMA,
        pltpu.SemaphoreType.DMA,
        pltpu.SemaphoreType.DMA,
        pltpu.SemaphoreType.DMA,
        pltpu.SemaphoreType.DMA,
        pltpu.SemaphoreType.DMA,
        pltpu.SemaphoreType.DMA,
    ],
)
def _edge(hx, srcs, dsts, embs, es_hbm, ee_hbm, zero_hbm, aggr,
          shacc, rows0, rows1, emb0, emb1, dst0, dst1, idx0, idx1, idx2,
          scat0, scat1, es_v, ee_v,
          sg0, sg1, sd0, sd1, se0, se1, si0, si1, si2, ss0, ss1):
  ROWS, EMBB, DSTB = (rows0, rows1), (emb0, emb1), (dst0, dst1)
  IDXB, SCATB = (idx0, idx1, idx2), (scat0, scat1)
  SG, SD, SE, SI, SS = (sg0, sg1), (sd0, sd1), (se0, se1), (si0, si1, si2), (ss0, ss1)
  wid = _wid()
  sid = lax.axis_index("s")
  tile_base = sid * RP
  pltpu.sync_copy(es_hbm, es_v)
  pltpu.sync_copy(ee_hbm, ee_v)

  for k in range(U // NW):
    u = wid + NW * k
    node_start = u * R
    pltpu.sync_copy(zero_hbm, shacc.at[pl.ds(tile_base, RP)])

    es = es_v[pl.ds(u, 16)][0]
    ee = ee_v[pl.ds(u, 16)][0]
    e_lo = es & ~7
    n_ch = (ee - e_lo + (CE - 1)) >> 7

    def cstart(i):
      return pl.multiple_of(e_lo + i * CE, 8)

    def issue_idx(i, q):
      pltpu.async_copy(srcs.at[pl.ds(cstart(i), CE)], IDXB[q], SI[q])

    def wait_idx(i, q):
      pltpu.make_async_copy(srcs.at[pl.ds(cstart(i), CE)], IDXB[q],
                            SI[q]).wait()

    def issue_lin(i, b):
      pltpu.async_copy(dsts.at[pl.ds(cstart(i), CE)], DSTB[b], SD[b])
      pltpu.async_copy(embs.at[pl.ds(cstart(i), CE)], EMBB[b], SE[b])

    def issue_gather(b, q):
      for hh in range(CE // 128):
        pltpu.async_copy(hx.at[IDXB[q].at[pl.ds(hh * 128, 128)]],
                         ROWS[b].at[pl.ds(hh * 128, 128)], SG[b])

    def wait_chunk(i, b):
      for hh in range(CE // 128):
        pltpu.make_async_copy(hx.at[IDXB[0].at[pl.ds(0, 128)]],
                              ROWS[b].at[pl.ds(hh * 128, 128)], SG[b]).wait()
      pltpu.make_async_copy(dsts.at[pl.ds(cstart(i), CE)], DSTB[b],
                            SD[b]).wait()
      pltpu.make_async_copy(embs.at[pl.ds(cstart(i), CE)], EMBB[b],
                            SE[b]).wait()

    def issue_scat(b):
      for hh in range(CE // 128):
        pltpu.async_copy(ROWS[b].at[pl.ds(hh * 128, 128)],
                         shacc.at[SCATB[b].at[hh]], SS[b], add=True)

    def wait_scat(b):
      for hh in range(CE // 128):
        pltpu.make_async_copy(ROWS[b].at[pl.ds(hh * 128, 128)],
                              shacc.at[SCATB[b].at[hh]], SS[b]).wait()

    @pl.when(n_ch >= 1)
    def _():
      issue_idx(0, 0)

    @pl.when(n_ch >= 2)
    def _():
      issue_idx(1, 1)

    @pl.when(n_ch >= 1)
    def _():
      wait_idx(0, 0)
      issue_gather(0, 0)
      issue_lin(0, 0)

    def outer(gg, carry):
      ibase = gg * 6
      for t in range(6):
        i = ibase + t
        b = t % 2
        b1 = (t + 1) % 2
        q1 = (t + 1) % 3
        q2 = (t + 2) % 3

        @pl.when(jnp.logical_and(i >= 1, i - 1 < n_ch))
        def _():
          wait_scat(b1)

        @pl.when(i + 1 < n_ch)
        def _():
          wait_idx(i + 1, q1)
          issue_gather(b1, q1)
          issue_lin(i + 1, b1)

        @pl.when(i + 2 < n_ch)
        def _():
          issue_idx(i + 2, q2)

        @pl.when(i < n_ch)
        def _():
          wait_chunk(i, b)
          for hh in range(CE // 128):
            def gidx(g, c2, hh=hh):
              dvec = DSTB[b][pl.ds(hh * 128 + g * 16, 16)] - node_start
              okv = jnp.logical_and(dvec >= 0, dvec < R)
              SCATB[b][hh, pl.ds(g * 16, 16)] = (
                  jnp.where(okv, dvec, R) + tile_base)
              return c2
            lax.fori_loop(0, 8, gidx, 0)

          def emsg(e, c2):
            for j in range(D // 16):
              sl = pl.ds(16 * j, 16)
              ROWS[b][e, sl] = jnp.maximum(ROWS[b][e, sl] + EMBB[b][e, sl],
                                           0.0)
            return c2

          lax.fori_loop(0, CE, emsg, 0)
          issue_scat(b)
      return carry

    lax.fori_loop(0, (n_ch + 1 + 5) // 6, outer, 0)
    pltpu.sync_copy(shacc.at[pl.ds(tile_base, R)],
                    aggr.at[pl.ds(pl.multiple_of(node_start, 8), R)])


@functools.partial(
    pl.kernel,
    out_type=jax.ShapeDtypeStruct((NG,), jnp.float32),
    mesh=_MESH,
    compiler_params=_SC_PARAMS,
    scratch_types=[
        pltpu.VMEM_SHARED((16 * 17, D), jnp.float32),
        pltpu.VMEM((17, D), jnp.float32),
        pltpu.VMEM((C, D), jnp.float32),
        pltpu.VMEM((D,), jnp.float32),
        pltpu.VMEM((16,), jnp.float32),
        pltpu.VMEM((C,), jnp.int32),
        pltpu.VMEM((C,), jnp.int32),
        pltpu.VMEM((NW + 16,), jnp.int32),
        pltpu.VMEM((NW + 16,), jnp.int32),
        pltpu.VMEM((16,), jnp.float32),
        pltpu.SemaphoreType.DMA,
    ],
)
def _pool(hfin, batchp, ns_hbm, ne_hbm, wp_hbm, bp_hbm, out,
          shacc, accum, rows_v, wp_v, out_v, b_v, scat_v, ns_v, ne_v, bp_v,
          sem):
  wid = _wid()
  sid = lax.axis_index("s")
  tile_base = sid * 17
  pltpu.sync_copy(ns_hbm, ns_v)
  pltpu.sync_copy(ne_hbm, ne_v)
  pltpu.sync_copy(bp_hbm, bp_v)
  pltpu.sync_copy(wp_hbm, wp_v)
  g0 = wid * 16
  zero16 = jnp.zeros((16,), jnp.float32)

  def zrow(r, carry):
    for j in range(D // 16):
      accum[r, pl.ds(16 * j, 16)] = zero16
    return carry

  lax.fori_loop(0, 17, zrow, 0)
  pltpu.sync_copy(accum, shacc.at[pl.ds(tile_base, 17)])

  n_lo = ns_v[pl.ds(wid, 16)][0] & ~7
  n_ch = (ne_v[pl.ds(wid, 16)][0] - n_lo + (C - 1)) >> 7

  def chunk(c, carry):
    cs = pl.multiple_of(n_lo + c * C, 8)
    pltpu.sync_copy(hfin.at[pl.ds(cs, C)], rows_v)
    pltpu.sync_copy(batchp.at[pl.ds(cs, C)], b_v)

    def gidx(g, carry2):
      gvec = b_v[pl.ds(g * 16, 16)] - g0
      okv = jnp.logical_and(gvec >= 0, gvec < 16)
      scat_v[pl.ds(g * 16, 16)] = jnp.where(okv, gvec, 16) + tile_base
      return carry2

    lax.fori_loop(0, C // 16, gidx, 0)
    pltpu.sync_copy(rows_v, shacc.at[scat_v], add=True)
    return carry

  lax.fori_loop(0, n_ch, chunk, 0)
  pltpu.sync_copy(shacc.at[pl.ds(tile_base, 17)], accum)

  lane = lax.iota(jnp.int32, 16)
  z = bp_v[...]
  for g in range(16):
    acc = jnp.zeros((16,), jnp.float32)
    for j in range(D // 16):
      sl = pl.ds(16 * j, 16)
      acc = acc + accum[g, sl] * wp_v[sl]
    tot = jnp.sum(acc)
    z = jnp.where(lane == g, z + tot, z)
  z = jnp.minimum(jnp.maximum(z, 0.0), 50.0)
  out_v[...] = z
  pltpu.sync_copy(out_v, out.at[pl.ds(pl.multiple_of(g0, 8), 16)])


_EB = 8192


def _emb_body(ea_ref, w_ref, b_ref, out_ref):
  out_ref[...] = (
      jnp.dot(ea_ref[...], w_ref[...], preferred_element_type=jnp.float32)
      + b_ref[0:1, :])


def _emb_call(ea_s, wep, bep):
  return pl.pallas_call(
      _emb_body,
      grid=(E_PAD // _EB,),
      in_specs=[
          pl.BlockSpec((_EB, KD), lambda i: (i, 0)),
          pl.BlockSpec((KD, D), lambda i: (0, 0)),
          pl.BlockSpec((8, D), lambda i: (0, 0)),
      ],
      out_specs=pl.BlockSpec((_EB, D), lambda i: (i, 0)),
      out_shape=jax.ShapeDtypeStruct((E_PAD, D), jnp.float32),
  )(ea_s, wep, bep)


def _p1_body(h_ref, a_ref, w1_ref, p_ref, t1_ref, s_ref):
  u = h_ref[...] * p_ref[4:5, :] + a_ref[...]
  t1 = jnp.dot(u, w1_ref[...], preferred_element_type=jnp.float32) + p_ref[0:1, :]
  t1_ref[...] = t1
  s1 = jnp.sum(t1, axis=0)
  s2 = jnp.sum(t1 * t1, axis=0)
  z = jnp.zeros((6, D), jnp.float32)
  s_ref[...] = jnp.concatenate([s1[None], s2[None], z], axis=0)[None]


def _p1_call(h, aggr, w1p, p):
  return pl.pallas_call(
      _p1_body,
      grid=(U,),
      in_specs=[
          pl.BlockSpec((R, D), lambda i: (i, 0)),
          pl.BlockSpec((R, D), lambda i: (i, 0)),
          pl.BlockSpec((D, D), lambda i: (0, 0)),
          pl.BlockSpec((8, D), lambda i: (0, 0)),
      ],
      out_specs=[
          pl.BlockSpec((R, D), lambda i: (i, 0)),
          pl.BlockSpec((1, 8, D), lambda i: (i, 0, 0)),
      ],
      out_shape=[
          jax.ShapeDtypeStruct((N2, D), jnp.float32),
          jax.ShapeDtypeStruct((U, 8, D), jnp.float32),
      ],
  )(h, aggr, w1p, p)


def _p2_body(last, t1_ref, h_ref, s_ref, w2_ref, p_ref, out_ref):
  i = pl.program_id(0)
  b1 = p_ref[0:1, :]
  s = s_ref[...]
  s1 = jnp.sum(s[:, 0, :], axis=0, keepdims=True) - float(N2 - N) * b1
  s2 = jnp.sum(s[:, 1, :], axis=0, keepdims=True) - float(N2 - N) * b1 * b1
  mu = s1 / float(N)
  var = s2 / float(N) - mu * mu
  inv = lax.rsqrt(var + 1e-5)
  t = (t1_ref[...] - mu) * inv * p_ref[1:2, :] + p_ref[2:3, :]
  t = jnp.maximum(t, 0.0)
  t = jnp.dot(t, w2_ref[...], preferred_element_type=jnp.float32) + p_ref[3:4, :]
  t = jnp.maximum(t, 0.0) * RS
  if not last:
    t = jnp.maximum(t, 0.0)
  hn = t + h_ref[...]
  rid = i * R + lax.broadcasted_iota(jnp.int32, (R, D), 0)
  out_ref[...] = jnp.where(rid < N, hn, 0.0)


def _p2_call(t1, h, s, w2p, p, last):
  return pl.pallas_call(
      functools.partial(_p2_body, last),
      grid=(U,),
      in_specs=[
          pl.BlockSpec((R, D), lambda i: (i, 0)),
          pl.BlockSpec((R, D), lambda i: (i, 0)),
          pl.BlockSpec((U, 8, D), lambda i: (0, 0, 0)),
          pl.BlockSpec((D, D), lambda i: (0, 0)),
          pl.BlockSpec((8, D), lambda i: (0, 0)),
      ],
      out_specs=pl.BlockSpec((R, D), lambda i: (i, 0)),
      out_shape=jax.ShapeDtypeStruct((N2, D), jnp.float32),
  )(t1, h, s, w2p, p)


def kernel(x, edge_index, edge_attr, batch, We, be, W1, b1, g1, bt1,
           W2, b2, eps, Wp, bp):
  f32 = jnp.float32
  src = edge_index[0].astype(jnp.int32)
  dst = edge_index[1].astype(jnp.int32)

  dst_s, order = lax.sort_key_val(dst, jnp.arange(E, dtype=jnp.int32))
  dst_sp = jnp.concatenate([dst_s, jnp.full((E_PAD - E,), N2, jnp.int32)])
  order_p = jnp.concatenate([order, jnp.zeros((E_PAD - E,), jnp.int32)])
  bounds = jnp.arange(U + 1, dtype=jnp.int32) * R
  ss = jnp.searchsorted(dst_s, bounds, side="left").astype(jnp.int32)
  es = jnp.pad(ss[:U], (0, 16))
  ee = jnp.pad(ss[1:], (0, 16))

  ea16 = jnp.pad(edge_attr.astype(f32), ((0, 0), (0, KD - 6)))
  src_s, ea_s = _prep(src, ea16, order_p)

  Wep = jnp.zeros((NUM_LAYERS, KD, D), f32).at[:, :6, :EMB].set(We)
  bep = jnp.zeros((NUM_LAYERS, 8, D), f32).at[:, 0, :EMB].set(be)
  W1p = jnp.zeros((NUM_LAYERS, D, D), f32).at[:, :EMB, :EMB].set(W1)
  W2p = jnp.zeros((NUM_LAYERS, D, D), f32).at[:, :EMB, :EMB].set(W2)
  P = jnp.zeros((NUM_LAYERS, 8, D), f32)
  P = P.at[:, 0, :EMB].set(b1)
  P = P.at[:, 1, :EMB].set(g1)
  P = P.at[:, 2, :EMB].set(bt1)
  P = P.at[:, 3, :EMB].set(b2)
  P = P.at[:, 4, :].set((1.0 + eps)[:, None])

  h = jnp.pad(x.astype(f32), ((0, N2 - N), (0, D - EMB)))
  zrows = jnp.zeros((RP, D), f32)

  for l in range(NUM_LAYERS):
    emb = _emb_call(ea_s, Wep[l], bep[l])
    aggr = _edge(h, src_s, dst_sp, emb, es, ee, zrows)
    t1, s = _p1_call(h, aggr, W1p[l], P[l])
    h = _p2_call(t1, h, s, W2p[l], P[l], l == NUM_LAYERS - 1)

  batchp = jnp.concatenate([batch.astype(jnp.int32),
                            jnp.full((N2 - N,), NG, jnp.int32)])
  gb = jnp.arange(NW + 1, dtype=jnp.int32) * 16
  nss = jnp.searchsorted(batch.astype(jnp.int32), gb, side="left").astype(jnp.int32)
  wpp = jnp.zeros((D,), f32).at[:EMB].set(Wp[:, 0])
  bp16 = jnp.broadcast_to(bp.astype(f32), (16,))
  out = _pool(h, batchp, jnp.pad(nss[:NW], (0, 16)), jnp.pad(nss[1:], (0, 16)),
              wpp, bp16)
  return out.reshape(NG, 1)


# speedup vs baseline: 2.7589x; 1.0449x over previous
import functools

import jax
import jax.numpy as jnp
from jax import lax
from jax.experimental import pallas as pl
from jax.experimental.pallas import tpu as pltpu
from jax.experimental.pallas import tpu_sc as plsc

NUM_LAYERS = 5
EMB = 81
D = 96
N = 50000
NG = 512
E = 800000
C = 128
U = 64
R = 784
N2 = U * R
E_PAD = 802816
NW = 32
EPT = E_PAD // NW
KD = 16
RS = 1.0 / (1.0 + 1e-5) ** 0.5

_MESH = plsc.VectorSubcoreMesh(
    core_axis_name="c", subcore_axis_name="s", num_cores=2, num_subcores=16)
_SC_PARAMS = pltpu.CompilerParams(use_tc_tiling_on_sc=False,
                                 needs_layout_passes=False)


def _wid():
  return lax.axis_index("s") * 2 + lax.axis_index("c")


PB = 896
NCHP = EPT // PB


@functools.partial(
    pl.kernel,
    out_type=(jax.ShapeDtypeStruct((E_PAD,), jnp.int32),
              jax.ShapeDtypeStruct((E_PAD, KD), jnp.float32)),
    mesh=_MESH,
    compiler_params=_SC_PARAMS,
    scratch_types=[
        pltpu.VMEM((PB,), jnp.int32),
        pltpu.VMEM((PB,), jnp.int32),
        pltpu.VMEM((PB,), jnp.int32),
        pltpu.VMEM((PB,), jnp.int32),
        pltpu.VMEM((PB, KD), jnp.float32),
        pltpu.VMEM((PB, KD), jnp.float32),
        pltpu.SemaphoreType.DMA,
        pltpu.SemaphoreType.DMA,
        pltpu.SemaphoreType.DMA,
        pltpu.SemaphoreType.DMA,
        pltpu.SemaphoreType.DMA,
        pltpu.SemaphoreType.DMA,
    ],
)
def _prep(src_hbm, ea_hbm, order_hbm, srcs_out, eas_out,
          oidx0, oidx1, sv0, sv1, eav0, eav1, so0, so1, sg0, sg1, sw0, sw1):
  OIDX, SV, EAV = (oidx0, oidx1), (sv0, sv1), (eav0, eav1)
  SO, SGA, SW = (so0, so1), (sg0, sg1), (sw0, sw1)
  base = _wid() * EPT

  def cstart(i):
    return pl.multiple_of(base + i * PB, 8)

  def issue_oidx(i, b):
    pltpu.async_copy(order_hbm.at[pl.ds(cstart(i), PB)], OIDX[b], SO[b])

  def wait_oidx(i, b):
    pltpu.make_async_copy(order_hbm.at[pl.ds(cstart(i), PB)], OIDX[b],
                          SO[b]).wait()

  def fire_gathers(b):
    for hh in range(PB // 128):
      sl = pl.ds(hh * 128, 128)
      pltpu.async_copy(src_hbm.at[OIDX[b].at[sl]], SV[b].at[sl], SGA[b])
      pltpu.async_copy(ea_hbm.at[OIDX[b].at[sl]], EAV[b].at[sl], SGA[b])

  def drain_gathers(b):
    for hh in range(PB // 128):
      sl = pl.ds(hh * 128, 128)
      pltpu.make_async_copy(src_hbm.at[OIDX[b].at[sl]], SV[b].at[sl],
                            SGA[b]).wait()
      pltpu.make_async_copy(ea_hbm.at[OIDX[b].at[sl]], EAV[b].at[sl],
                            SGA[b]).wait()

  def issue_writes(i, b):
    pltpu.async_copy(SV[b], srcs_out.at[pl.ds(cstart(i), PB)], SW[b])
    pltpu.async_copy(EAV[b], eas_out.at[pl.ds(cstart(i), PB)], SW[b])

  def wait_writes(i, b):
    pltpu.make_async_copy(SV[b], srcs_out.at[pl.ds(cstart(i), PB)],
                          SW[b]).wait()
    pltpu.make_async_copy(EAV[b], eas_out.at[pl.ds(cstart(i), PB)],
                          SW[b]).wait()

  issue_oidx(0, 0)

  def step(i, carry):
    for t in range(2):
      ii = i * 2 + t

      @pl.when(ii + 1 < NCHP)
      def _():
        issue_oidx(ii + 1, (t + 1) % 2)

      @pl.when(ii < NCHP)
      def _():
        wait_oidx(ii, t)

      @pl.when(ii >= 2)
      def _():
        wait_writes(ii - 2, t)

      @pl.when(ii < NCHP)
      def _():
        fire_gathers(t)
        drain_gathers(t)
        issue_writes(ii, t)
    return carry

  lax.fori_loop(0, NCHP // 2, step, 0)
  wait_writes(NCHP - 2, 0)
  wait_writes(NCHP - 1, 1)


RP = R + 1
ZR = 157
CE = 128


@functools.partial(
    pl.kernel,
    out_type=jax.ShapeDtypeStruct((N2, D), jnp.float32),
    mesh=_MESH,
    compiler_params=_SC_PARAMS,
    scratch_types=[
        pltpu.VMEM_SHARED((16 * RP, D), jnp.float32),
        pltpu.VMEM((CE, D), jnp.float32),
        pltpu.VMEM((CE, D), jnp.float32),
        pltpu.VMEM((CE, D), jnp.float32),
        pltpu.VMEM((CE, D), jnp.float32),
        pltpu.VMEM((CE,), jnp.int32),
        pltpu.VMEM((CE,), jnp.int32),
        pltpu.VMEM((CE,), jnp.int32),
        pltpu.VMEM((CE,), jnp.int32),
        pltpu.VMEM((CE,), jnp.int32),
        pltpu.VMEM((2, 128), jnp.int32),
        pltpu.VMEM((2, 128), jnp.int32),
        pltpu.VMEM((U + 16,), jnp.int32),
        pltpu.VMEM((U + 16,), jnp.int32),
        pltpu.SemaphoreType.DMA,
        pltpu.SemaphoreType.DMA,
        pltpu.SemaphoreType.DMA,
        pltpu.SemaphoreType.DMA,
        pltpu.SemaphoreType.DMA,
        pltpu.SemaphoreType.DMA,
        pltpu.SemaphoreType.DMA,
        pltpu.SemaphoreType.DMA,
        pltpu.SemaphoreType.DMA,
        pltpu.SemaphoreType.DMA,
        pltpu.SemaphoreType.DMA,
    ],
)
def _edge(hx, srcs, dsts, embs, es_hbm, ee_hbm, zero_hbm, aggr,
          shacc, rows0, rows1, emb0, emb1, dst0, dst1, idx0, idx1, idx2,
          scat0, scat1, es_v, ee_v,
          sg0, sg1, sd0, sd1, se0, se1, si0, si1, si2, ss0, ss1):
  ROWS, EMBB, DSTB = (rows0, rows1), (emb0, emb1), (dst0, dst1)
  IDXB, SCATB = (idx0, idx1, idx2), (scat0, scat1)
  SG, SD, SE, SI, SS = (sg0, sg1), (sd0, sd1), (se0, se1), (si0, si1, si2), (ss0, ss1)
  wid = _wid()
  sid = lax.axis_index("s")
  tile_base = sid * RP
  pltpu.sync_copy(es_hbm, es_v)
  pltpu.sync_copy(ee_hbm, ee_v)

  for k in range(U // NW):
    u = wid + NW * k
    node_start = u * R
    pltpu.sync_copy(zero_hbm, shacc.at[pl.ds(tile_base, RP)])

    es = es_v[pl.ds(u, 16)][0]
    ee = ee_v[pl.ds(u, 16)][0]
    e_lo = es & ~7
    n_ch = (ee - e_lo + (CE - 1)) >> 7

    def cstart(i):
      return pl.multiple_of(e_lo + i * CE, 8)

    def issue_idx(i, q):
      pltpu.async_copy(srcs.at[pl.ds(cstart(i), CE)], IDXB[q], SI[q])

    def wait_idx(i, q):
      pltpu.make_async_copy(srcs.at[pl.ds(cstart(i), CE)], IDXB[q],
                            SI[q]).wait()

    def issue_lin(i, b):
      pltpu.async_copy(dsts.at[pl.ds(cstart(i), CE)], DSTB[b], SD[b])
      pltpu.async_copy(embs.at[pl.ds(cstart(i), CE)], EMBB[b], SE[b])

    def issue_gather(b, q):
      for hh in range(CE // 128):
        pltpu.async_copy(hx.at[IDXB[q].at[pl.ds(hh * 128, 128)]],
                         ROWS[b].at[pl.ds(hh * 128, 128)], SG[b])

    def wait_chunk(i, b):
      for hh in range(CE // 128):
        pltpu.make_async_copy(hx.at[IDXB[0].at[pl.ds(0, 128)]],
                              ROWS[b].at[pl.ds(hh * 128, 128)], SG[b]).wait()
      pltpu.make_async_copy(dsts.at[pl.ds(cstart(i), CE)], DSTB[b],
                            SD[b]).wait()
      pltpu.make_async_copy(embs.at[pl.ds(cstart(i), CE)], EMBB[b],
                            SE[b]).wait()

    def issue_scat(b):
      for hh in range(CE // 128):
        pltpu.async_copy(ROWS[b].at[pl.ds(hh * 128, 128)],
                         shacc.at[SCATB[b].at[hh]], SS[b], add=True)

    def wait_scat(b):
      for hh in range(CE // 128):
        pltpu.make_async_copy(ROWS[b].at[pl.ds(hh * 128, 128)],
                              shacc.at[SCATB[b].at[hh]], SS[b]).wait()

    @pl.when(n_ch >= 1)
    def _():
      issue_idx(0, 0)

    @pl.when(n_ch >= 2)
    def _():
      issue_idx(1, 1)

    @pl.when(n_ch >= 1)
    def _():
      wait_idx(0, 0)
      issue_gather(0, 0)
      issue_lin(0, 0)

    def outer(gg, carry):
      ibase = gg * 6
      for t in range(6):
        i = ibase + t
        b = t % 2
        b1 = (t + 1) % 2
        q1 = (t + 1) % 3
        q2 = (t + 2) % 3

        @pl.when(jnp.logical_and(i >= 1, i - 1 < n_ch))
        def _():
          wait_scat(b1)

        @pl.when(i + 1 < n_ch)
        def _():
          wait_idx(i + 1, q1)
          issue_gather(b1, q1)
          issue_lin(i + 1, b1)

        @pl.when(i + 2 < n_ch)
        def _():
          issue_idx(i + 2, q2)

        @pl.when(i < n_ch)
        def _():
          wait_chunk(i, b)
          for hh in range(CE // 128):
            def gidx(g, c2, hh=hh):
              dvec = DSTB[b][pl.ds(hh * 128 + g * 16, 16)] - node_start
              okv = jnp.logical_and(dvec >= 0, dvec < R)
              SCATB[b][hh, pl.ds(g * 16, 16)] = (
                  jnp.where(okv, dvec, R) + tile_base)
              return c2
            lax.fori_loop(0, 8, gidx, 0)

          def emsg(e, c2):
            for j in range(D // 16):
              sl = pl.ds(16 * j, 16)
              ROWS[b][e, sl] = jnp.maximum(ROWS[b][e, sl] + EMBB[b][e, sl],
                                           0.0)
            return c2

          lax.fori_loop(0, CE, emsg, 0)
          issue_scat(b)
      return carry

    lax.fori_loop(0, (n_ch + 1 + 5) // 6, outer, 0)
    pltpu.sync_copy(shacc.at[pl.ds(tile_base, R)],
                    aggr.at[pl.ds(pl.multiple_of(node_start, 8), R)])


@functools.partial(
    pl.kernel,
    out_type=jax.ShapeDtypeStruct((NG,), jnp.float32),
    mesh=_MESH,
    compiler_params=_SC_PARAMS,
    scratch_types=[
        pltpu.VMEM_SHARED((16 * 17, D), jnp.float32),
        pltpu.VMEM((17, D), jnp.float32),
        pltpu.VMEM((C, D), jnp.float32),
        pltpu.VMEM((D,), jnp.float32),
        pltpu.VMEM((16,), jnp.float32),
        pltpu.VMEM((C,), jnp.int32),
        pltpu.VMEM((C,), jnp.int32),
        pltpu.VMEM((NW + 16,), jnp.int32),
        pltpu.VMEM((NW + 16,), jnp.int32),
        pltpu.VMEM((16,), jnp.float32),
        pltpu.SemaphoreType.DMA,
    ],
)
def _pool(hfin, batchp, ns_hbm, ne_hbm, wp_hbm, bp_hbm, out,
          shacc, accum, rows_v, wp_v, out_v, b_v, scat_v, ns_v, ne_v, bp_v,
          sem):
  wid = _wid()
  sid = lax.axis_index("s")
  tile_base = sid * 17
  pltpu.sync_copy(ns_hbm, ns_v)
  pltpu.sync_copy(ne_hbm, ne_v)
  pltpu.sync_copy(bp_hbm, bp_v)
  pltpu.sync_copy(wp_hbm, wp_v)
  g0 = wid * 16
  zero16 = jnp.zeros((16,), jnp.float32)

  def zrow(r, carry):
    for j in range(D // 16):
      accum[r, pl.ds(16 * j, 16)] = zero16
    return carry

  lax.fori_loop(0, 17, zrow, 0)
  pltpu.sync_copy(accum, shacc.at[pl.ds(tile_base, 17)])

  n_lo = ns_v[pl.ds(wid, 16)][0] & ~7
  n_ch = (ne_v[pl.ds(wid, 16)][0] - n_lo + (C - 1)) >> 7

  def chunk(c, carry):
    cs = pl.multiple_of(n_lo + c * C, 8)
    pltpu.sync_copy(hfin.at[pl.ds(cs, C)], rows_v)
    pltpu.sync_copy(batchp.at[pl.ds(cs, C)], b_v)

    def gidx(g, carry2):
      gvec = b_v[pl.ds(g * 16, 16)] - g0
      okv = jnp.logical_and(gvec >= 0, gvec < 16)
      scat_v[pl.ds(g * 16, 16)] = jnp.where(okv, gvec, 16) + tile_base
      return carry2

    lax.fori_loop(0, C // 16, gidx, 0)
    pltpu.sync_copy(rows_v, shacc.at[scat_v], add=True)
    return carry

  lax.fori_loop(0, n_ch, chunk, 0)
  pltpu.sync_copy(shacc.at[pl.ds(tile_base, 17)], accum)

  lane = lax.iota(jnp.int32, 16)
  z = bp_v[...]
  for g in range(16):
    acc = jnp.zeros((16,), jnp.float32)
    for j in range(D // 16):
      sl = pl.ds(16 * j, 16)
      acc = acc + accum[g, sl] * wp_v[sl]
    tot = jnp.sum(acc)
    z = jnp.where(lane == g, z + tot, z)
  z = jnp.minimum(jnp.maximum(z, 0.0), 50.0)
  out_v[...] = z
  pltpu.sync_copy(out_v, out.at[pl.ds(pl.multiple_of(g0, 8), 16)])


_EB = 8192


def _emb_body(ea_ref, w_ref, b_ref, out_ref):
  out_ref[...] = (
      jnp.dot(ea_ref[...], w_ref[...], preferred_element_type=jnp.float32)
      + b_ref[0:1, :])


def _emb_call(ea_s, wep, bep):
  return pl.pallas_call(
      _emb_body,
      grid=(E_PAD // _EB,),
      in_specs=[
          pl.BlockSpec((_EB, KD), lambda i: (i, 0)),
          pl.BlockSpec((KD, D), lambda i: (0, 0)),
          pl.BlockSpec((8, D), lambda i: (0, 0)),
      ],
      out_specs=pl.BlockSpec((_EB, D), lambda i: (i, 0)),
      out_shape=jax.ShapeDtypeStruct((E_PAD, D), jnp.float32),
  )(ea_s, wep, bep)


def _p1_body(h_ref, a_ref, w1_ref, p_ref, t1_ref, s_ref):
  u = h_ref[...] * p_ref[4:5, :] + a_ref[...]
  t1 = jnp.dot(u, w1_ref[...], preferred_element_type=jnp.float32) + p_ref[0:1, :]
  t1_ref[...] = t1
  s1 = jnp.sum(t1, axis=0)
  s2 = jnp.sum(t1 * t1, axis=0)
  z = jnp.zeros((6, D), jnp.float32)
  s_ref[...] = jnp.concatenate([s1[None], s2[None], z], axis=0)[None]


def _p1_call(h, aggr, w1p, p):
  return pl.pallas_call(
      _p1_body,
      grid=(U,),
      in_specs=[
          pl.BlockSpec((R, D), lambda i: (i, 0)),
          pl.BlockSpec((R, D), lambda i: (i, 0)),
          pl.BlockSpec((D, D), lambda i: (0, 0)),
          pl.BlockSpec((8, D), lambda i: (0, 0)),
      ],
      out_specs=[
          pl.BlockSpec((R, D), lambda i: (i, 0)),
          pl.BlockSpec((1, 8, D), lambda i: (i, 0, 0)),
      ],
      out_shape=[
          jax.ShapeDtypeStruct((N2, D), jnp.float32),
          jax.ShapeDtypeStruct((U, 8, D), jnp.float32),
      ],
  )(h, aggr, w1p, p)


def _p2_body(last, t1_ref, h_ref, s_ref, w2_ref, p_ref, out_ref):
  i = pl.program_id(0)
  b1 = p_ref[0:1, :]
  s = s_ref[...]
  s1 = jnp.sum(s[:, 0, :], axis=0, keepdims=True) - float(N2 - N) * b1
  s2 = jnp.sum(s[:, 1, :], axis=0, keepdims=True) - float(N2 - N) * b1 * b1
  mu = s1 / float(N)
  var = s2 / float(N) - mu * mu
  inv = lax.rsqrt(var + 1e-5)
  t = (t1_ref[...] - mu) * inv * p_ref[1:2, :] + p_ref[2:3, :]
  t = jnp.maximum(t, 0.0)
  t = jnp.dot(t, w2_ref[...], preferred_element_type=jnp.float32) + p_ref[3:4, :]
  t = jnp.maximum(t, 0.0) * RS
  if not last:
    t = jnp.maximum(t, 0.0)
  hn = t + h_ref[...]
  rid = i * R + lax.broadcasted_iota(jnp.int32, (R, D), 0)
  out_ref[...] = jnp.where(rid < N, hn, 0.0)


def _p2_call(t1, h, s, w2p, p, last):
  return pl.pallas_call(
      functools.partial(_p2_body, last),
      grid=(U,),
      in_specs=[
          pl.BlockSpec((R, D), lambda i: (i, 0)),
          pl.BlockSpec((R, D), lambda i: (i, 0)),
          pl.BlockSpec((U, 8, D), lambda i: (0, 0, 0)),
          pl.BlockSpec((D, D), lambda i: (0, 0)),
          pl.BlockSpec((8, D), lambda i: (0, 0)),
      ],
      out_specs=pl.BlockSpec((R, D), lambda i: (i, 0)),
      out_shape=jax.ShapeDtypeStruct((N2, D), jnp.float32),
  )(t1, h, s, w2p, p)


def kernel(x, edge_index, edge_attr, batch, We, be, W1, b1, g1, bt1,
           W2, b2, eps, Wp, bp):
  f32 = jnp.float32
  src = edge_index[0].astype(jnp.int32)
  dst = edge_index[1].astype(jnp.int32)

  dst_s, order = lax.sort_key_val(dst, jnp.arange(E, dtype=jnp.int32))
  dst_sp = jnp.concatenate([dst_s, jnp.full((E_PAD - E,), N2, jnp.int32)])
  order_p = jnp.concatenate([order, jnp.zeros((E_PAD - E,), jnp.int32)])
  bounds = jnp.arange(U + 1, dtype=jnp.int32) * R
  ss = jnp.searchsorted(dst_s, bounds, side="left").astype(jnp.int32)
  es = jnp.pad(ss[:U], (0, 16))
  ee = jnp.pad(ss[1:], (0, 16))

  ea16 = jnp.pad(edge_attr.astype(f32), ((0, 0), (0, KD - 6)))
  src_s, ea_s = _prep(src, ea16, order_p)

  Wep = jnp.zeros((NUM_LAYERS, KD, D), f32).at[:, :6, :EMB].set(We)
  bep = jnp.zeros((NUM_LAYERS, 8, D), f32).at[:, 0, :EMB].set(be)
  W1p = jnp.zeros((NUM_LAYERS, D, D), f32).at[:, :EMB, :EMB].set(W1)
  W2p = jnp.zeros((NUM_LAYERS, D, D), f32).at[:, :EMB, :EMB].set(W2)
  P = jnp.zeros((NUM_LAYERS, 8, D), f32)
  P = P.at[:, 0, :EMB].set(b1)
  P = P.at[:, 1, :EMB].set(g1)
  P = P.at[:, 2, :EMB].set(bt1)
  P = P.at[:, 3, :EMB].set(b2)
  P = P.at[:, 4, :].set((1.0 + eps)[:, None])

  h = jnp.pad(x.astype(f32), ((0, N2 - N), (0, D - EMB)))
  zrows = jnp.zeros((RP, D), f32)

  embs_all = [_emb_call(ea_s, Wep[l], bep[l]) for l in range(NUM_LAYERS)]
  for l in range(NUM_LAYERS):
    aggr = _edge(h, src_s, dst_sp, embs_all[l], es, ee, zrows)
    t1, s = _p1_call(h, aggr, W1p[l], P[l])
    h = _p2_call(t1, h, s, W2p[l], P[l], l == NUM_LAYERS - 1)

  batchp = jnp.concatenate([batch.astype(jnp.int32),
                            jnp.full((N2 - N,), NG, jnp.int32)])
  gb = jnp.arange(NW + 1, dtype=jnp.int32) * 16
  nss = jnp.searchsorted(batch.astype(jnp.int32), gb, side="left").astype(jnp.int32)
  wpp = jnp.zeros((D,), f32).at[:EMB].set(Wp[:, 0])
  bp16 = jnp.broadcast_to(bp.astype(f32), (16,))
  out = _pool(h, batchp, jnp.pad(nss[:NW], (0, 16)), jnp.pad(nss[1:], (0, 16)),
              wpp, bp16)
  return out.reshape(NG, 1)
